# Initial kernel scaffold; baseline (speedup 1.0000x reference)
import jax
import jax.numpy as jnp
from jax.experimental import pallas as pl


def kernel(x, edge_index, edge_attr, Wu, bu, Wv, bv, WA, bA, WB, bB, WC, bC, gamma_node, beta_node, gamma_edge, beta_edge):
    raise NotImplementedError("write your pallas kernel here")


# speedup vs baseline: 1.0230x; 1.0230x over previous
import functools

import jax
import jax.numpy as jnp
from jax.experimental import pallas as pl
from jax.experimental.pallas import tpu as pltpu

N, E, D = 10000, 320000, 128
TE = 2000
GRID_E = E // TE


def _tables_body(x_ref, wu_ref, bu_ref, wv_ref, bv_ref, wb_ref, bb_ref,
                 wc_ref, bc_ref, xu_ref, xv_ref, xb_ref, xc_ref):
    xx = x_ref[...]
    xu_ref[...] = jnp.dot(xx, wu_ref[...], preferred_element_type=jnp.float32) + bu_ref[...]
    xv_ref[...] = jnp.dot(xx, wv_ref[...], preferred_element_type=jnp.float32) + bv_ref[...]
    xb_ref[...] = jnp.dot(xx, wb_ref[...], preferred_element_type=jnp.float32) + bb_ref[...]
    xc_ref[...] = jnp.dot(xx, wc_ref[...], preferred_element_type=jnp.float32) + bc_ref[...]


def _tables(x, WuT, bu, WvT, bv, WBT, bB, WCT, bC):
    out = jax.ShapeDtypeStruct((N, D), jnp.float32)
    return pl.pallas_call(
        _tables_body,
        out_shape=(out, out, out, out),
    )(x, WuT, bu, WvT, bv, WBT, bB, WCT, bC)


def _stats_body(ea_ref, gsum_ref, wat_ref, ba_ref, s1_ref, s2_ref, acc1, acc2):
    i = pl.program_id(0)

    @pl.when(i == 0)
    def _():
        acc1[...] = jnp.zeros_like(acc1)
        acc2[...] = jnp.zeros_like(acc2)

    ein = (jnp.dot(ea_ref[...], wat_ref[...], preferred_element_type=jnp.float32)
           + ba_ref[...] + gsum_ref[...])
    e3 = ein.reshape(TE // 8, 8, D)
    acc1[...] += jnp.sum(e3, axis=0)
    acc2[...] += jnp.sum(e3 * e3, axis=0)

    @pl.when(i == GRID_E - 1)
    def _():
        s1_ref[...] = acc1[...]
        s2_ref[...] = acc2[...]


def _edge_stats(edge_attr, gsum, WAT, bA):
    s = jax.ShapeDtypeStruct((8, D), jnp.float32)
    return pl.pallas_call(
        _stats_body,
        grid=(GRID_E,),
        in_specs=[
            pl.BlockSpec((TE, D), lambda i: (i, 0)),
            pl.BlockSpec((TE, D), lambda i: (i, 0)),
            pl.BlockSpec((D, D), lambda i: (0, 0)),
            pl.BlockSpec((1, D), lambda i: (0, 0)),
        ],
        out_specs=(pl.BlockSpec((8, D), lambda i: (0, 0)),
                   pl.BlockSpec((8, D), lambda i: (0, 0))),
        out_shape=(s, s),
        scratch_shapes=[pltpu.VMEM((8, D), jnp.float32),
                        pltpu.VMEM((8, D), jnp.float32)],
    )(edge_attr, gsum, WAT, bA)


def _apply_body(ea_ref, gsum_ref, gxv_ref, wat_ref, ba_ref, s1_ref, s2_ref,
                ge_ref, be_ref, eout_ref, msg_ref):
    s1 = jnp.sum(s1_ref[...], axis=0, keepdims=True)
    s2 = jnp.sum(s2_ref[...], axis=0, keepdims=True)
    mean = s1 / E
    var = s2 / E - mean * mean
    rstd = jax.lax.rsqrt(var + 1e-5)
    ein = (jnp.dot(ea_ref[...], wat_ref[...], preferred_element_type=jnp.float32)
           + ba_ref[...] + gsum_ref[...])
    tmp = jnp.maximum(ge_ref[...] * (ein - mean) * rstd + be_ref[...], 0.0)
    eout = ea_ref[...] + tmp
    eout_ref[...] = eout
    msg_ref[...] = jax.nn.sigmoid(eout) * gxv_ref[...]


def _edge_apply(edge_attr, gsum, gxv, WAT, bA, s1, s2, gamma_e, beta_e):
    out = jax.ShapeDtypeStruct((E, D), jnp.float32)
    return pl.pallas_call(
        _apply_body,
        grid=(GRID_E,),
        in_specs=[
            pl.BlockSpec((TE, D), lambda i: (i, 0)),
            pl.BlockSpec((TE, D), lambda i: (i, 0)),
            pl.BlockSpec((TE, D), lambda i: (i, 0)),
            pl.BlockSpec((D, D), lambda i: (0, 0)),
            pl.BlockSpec((1, D), lambda i: (0, 0)),
            pl.BlockSpec((8, D), lambda i: (0, 0)),
            pl.BlockSpec((8, D), lambda i: (0, 0)),
            pl.BlockSpec((1, D), lambda i: (0, 0)),
            pl.BlockSpec((1, D), lambda i: (0, 0)),
        ],
        out_specs=(pl.BlockSpec((TE, D), lambda i: (i, 0)),
                   pl.BlockSpec((TE, D), lambda i: (i, 0))),
        out_shape=(out, out),
    )(edge_attr, gsum, gxv, WAT, bA, s1, s2, gamma_e, beta_e)


def _node_body(x_ref, xu_ref, agg_ref, gn_ref, bn_ref, xo_ref):
    node_in = xu_ref[...] + agg_ref[...]
    mean = jnp.mean(node_in, axis=0, keepdims=True)
    var = jnp.mean(node_in * node_in, axis=0, keepdims=True) - mean * mean
    rstd = jax.lax.rsqrt(var + 1e-5)
    tmp = jnp.maximum(gn_ref[...] * (node_in - mean) * rstd + bn_ref[...], 0.0)
    xo_ref[...] = x_ref[...] + tmp


def _node_final(x, xu, agg, gamma_n, beta_n):
    return pl.pallas_call(
        _node_body,
        out_shape=jax.ShapeDtypeStruct((N, D), jnp.float32),
    )(x, xu, agg, gamma_n, beta_n)


def kernel(x, edge_index, edge_attr, Wu, bu, Wv, bv, WA, bA, WB, bB, WC, bC,
           gamma_node, beta_node, gamma_edge, beta_edge):
    row = edge_index[0]
    col = edge_index[1]
    bu2 = bu.reshape(1, D)
    bv2 = bv.reshape(1, D)
    bA2 = bA.reshape(1, D)
    bB2 = bB.reshape(1, D)
    bC2 = bC.reshape(1, D)
    ge2 = gamma_edge.reshape(1, D)
    be2 = beta_edge.reshape(1, D)
    gn2 = gamma_node.reshape(1, D)
    bn2 = beta_node.reshape(1, D)

    xu, xv, xB, xC = _tables(x, Wu.T, bu2, Wv.T, bv2, WB.T, bB2, WC.T, bC2)

    gsum = jnp.take(xB, row, axis=0) + jnp.take(xC, col, axis=0)
    gxv = jnp.take(xv, col, axis=0)

    s1, s2 = _edge_stats(edge_attr, gsum, WA.T, bA2)
    eout, msg = _edge_apply(edge_attr, gsum, gxv, WA.T, bA2, s1, s2, ge2, be2)

    agg = jax.ops.segment_sum(msg, row, num_segments=N)

    x_out = _node_final(x, xu, agg, gn2, bn2)
    return (x_out, eout)


# speedup vs baseline: 3.4554x; 3.3778x over previous
import functools

import jax
import jax.numpy as jnp
from jax import lax
from jax.experimental import pallas as pl
from jax.experimental.pallas import tpu as pltpu
from jax.experimental.pallas import tpu_sc as plsc

N, E, D = 10000, 320000, 128
TE = 2000
GRID_E = E // TE

NW = 32
EW = E // NW
KG = 400
KS = 200
NSTRIPE = 632


def _sc_gather(xB, xC, xv, row, col):
    mesh = plsc.VectorSubcoreMesh(core_axis_name="c", subcore_axis_name="s")
    out = jax.ShapeDtypeStruct((E, D), jnp.float32)

    @functools.partial(
        pl.kernel, mesh=mesh, out_type=(out, out),
        scratch_types=[
            pltpu.VMEM((KG,), jnp.int32),
            pltpu.VMEM((KG,), jnp.int32),
            pltpu.VMEM((KG, D), jnp.float32),
            pltpu.VMEM((KG, D), jnp.float32),
            pltpu.SemaphoreType.DMA,
            pltpu.SemaphoreType.DMA,
        ])
    def k(xB_hbm, xC_hbm, xv_hbm, row_hbm, col_hbm, gsum_hbm, gxv_hbm,
          rowi_v, coli_v, bufB_v, bufC_v, sem1, sem2):
        wid = lax.axis_index("s") * 2 + lax.axis_index("c")
        base0 = wid * EW

        @pl.loop(0, EW, step=KG)
        def _(off):
            base = base0 + off
            pltpu.sync_copy(row_hbm.at[pl.ds(base, KG)], rowi_v)
            pltpu.sync_copy(col_hbm.at[pl.ds(base, KG)], coli_v)
            cpB = pltpu.async_copy(xB_hbm.at[rowi_v], bufB_v, sem1)
            cpC = pltpu.async_copy(xC_hbm.at[coli_v], bufC_v, sem2)
            cpB.wait()
            cpC.wait()

            @pl.loop(0, KG)
            def _(r):
                for j in range(8):
                    plsc.addupdate(bufB_v.at[r, pl.ds(j * 16, 16)],
                                   bufC_v[r, pl.ds(j * 16, 16)])

            cpV = pltpu.async_copy(xv_hbm.at[coli_v], bufC_v, sem2)
            pltpu.sync_copy(bufB_v, gsum_hbm.at[pl.ds(base, KG)])
            cpV.wait()
            pltpu.sync_copy(bufC_v, gxv_hbm.at[pl.ds(base, KG)])

    return k(xB, xC, xv, row, col)


def _sc_scatter(msg, row, zeros_nd):
    mesh = plsc.VectorSubcoreMesh(core_axis_name="c", subcore_axis_name="s")
    out = jax.ShapeDtypeStruct((2, N, D), jnp.float32)

    @functools.partial(
        pl.kernel, mesh=mesh, out_type=out,
        scratch_types=[
            pltpu.VMEM((KS,), jnp.int32),
            pltpu.VMEM((KS, D), jnp.float32),
            pltpu.VMEM_SHARED((N, D), jnp.float32),
        ])
    def k(msg_hbm, row_hbm, zero_hbm, agg_hbm, rowi_v, msg_v, acc_sh):
        cid = lax.axis_index("c")
        sid = lax.axis_index("s")
        wid = sid * 2 + cid
        base0 = wid * EW

        @pl.when(sid == 0)
        def _():
            pltpu.sync_copy(zero_hbm, acc_sh)

        plsc.subcore_barrier()

        @pl.loop(0, EW, step=KS)
        def _(off):
            base = base0 + off
            pltpu.sync_copy(row_hbm.at[pl.ds(base, KS)], rowi_v)
            pltpu.sync_copy(msg_hbm.at[pl.ds(base, KS)], msg_v)
            pltpu.sync_copy(msg_v, acc_sh.at[rowi_v], add=True)

        plsc.subcore_barrier()

        @pl.when(sid < 15)
        def _():
            pltpu.sync_copy(acc_sh.at[pl.ds(sid * NSTRIPE, NSTRIPE)],
                            agg_hbm.at[cid].at[pl.ds(sid * NSTRIPE, NSTRIPE)])

        @pl.when(sid == 15)
        def _():
            pltpu.sync_copy(acc_sh.at[pl.ds(15 * NSTRIPE, N - 15 * NSTRIPE)],
                            agg_hbm.at[cid].at[pl.ds(15 * NSTRIPE, N - 15 * NSTRIPE)])

    return k(msg, row, zeros_nd)


def _tables_body(x_ref, wu_ref, bu_ref, wv_ref, bv_ref, wb_ref, bb_ref,
                 wc_ref, bc_ref, xu_ref, xv_ref, xb_ref, xc_ref):
    xx = x_ref[...]
    xu_ref[...] = jnp.dot(xx, wu_ref[...], preferred_element_type=jnp.float32) + bu_ref[...]
    xv_ref[...] = jnp.dot(xx, wv_ref[...], preferred_element_type=jnp.float32) + bv_ref[...]
    xb_ref[...] = jnp.dot(xx, wb_ref[...], preferred_element_type=jnp.float32) + bb_ref[...]
    xc_ref[...] = jnp.dot(xx, wc_ref[...], preferred_element_type=jnp.float32) + bc_ref[...]


def _tables(x, WuT, bu, WvT, bv, WBT, bB, WCT, bC):
    out = jax.ShapeDtypeStruct((N, D), jnp.float32)
    return pl.pallas_call(
        _tables_body,
        out_shape=(out, out, out, out),
    )(x, WuT, bu, WvT, bv, WBT, bB, WCT, bC)


def _stats_body(ea_ref, gsum_ref, wat_ref, ba_ref, s1_ref, s2_ref, acc1, acc2):
    i = pl.program_id(0)

    @pl.when(i == 0)
    def _():
        acc1[...] = jnp.zeros_like(acc1)
        acc2[...] = jnp.zeros_like(acc2)

    ein = (jnp.dot(ea_ref[...], wat_ref[...], preferred_element_type=jnp.float32)
           + ba_ref[...] + gsum_ref[...])
    e3 = ein.reshape(TE // 8, 8, D)
    acc1[...] += jnp.sum(e3, axis=0)
    acc2[...] += jnp.sum(e3 * e3, axis=0)

    @pl.when(i == GRID_E - 1)
    def _():
        s1_ref[...] = acc1[...]
        s2_ref[...] = acc2[...]


def _edge_stats(edge_attr, gsum, WAT, bA):
    s = jax.ShapeDtypeStruct((8, D), jnp.float32)
    return pl.pallas_call(
        _stats_body,
        grid=(GRID_E,),
        in_specs=[
            pl.BlockSpec((TE, D), lambda i: (i, 0)),
            pl.BlockSpec((TE, D), lambda i: (i, 0)),
            pl.BlockSpec((D, D), lambda i: (0, 0)),
            pl.BlockSpec((1, D), lambda i: (0, 0)),
        ],
        out_specs=(pl.BlockSpec((8, D), lambda i: (0, 0)),
                   pl.BlockSpec((8, D), lambda i: (0, 0))),
        out_shape=(s, s),
        scratch_shapes=[pltpu.VMEM((8, D), jnp.float32),
                        pltpu.VMEM((8, D), jnp.float32)],
    )(edge_attr, gsum, WAT, bA)


def _apply_body(ea_ref, gsum_ref, gxv_ref, wat_ref, ba_ref, s1_ref, s2_ref,
                ge_ref, be_ref, eout_ref, msg_ref):
    s1 = jnp.sum(s1_ref[...], axis=0, keepdims=True)
    s2 = jnp.sum(s2_ref[...], axis=0, keepdims=True)
    mean = s1 / E
    var = s2 / E - mean * mean
    rstd = jax.lax.rsqrt(var + 1e-5)
    ein = (jnp.dot(ea_ref[...], wat_ref[...], preferred_element_type=jnp.float32)
           + ba_ref[...] + gsum_ref[...])
    tmp = jnp.maximum(ge_ref[...] * (ein - mean) * rstd + be_ref[...], 0.0)
    eout = ea_ref[...] + tmp
    eout_ref[...] = eout
    msg_ref[...] = jax.nn.sigmoid(eout) * gxv_ref[...]


def _edge_apply(edge_attr, gsum, gxv, WAT, bA, s1, s2, gamma_e, beta_e):
    out = jax.ShapeDtypeStruct((E, D), jnp.float32)
    return pl.pallas_call(
        _apply_body,
        grid=(GRID_E,),
        in_specs=[
            pl.BlockSpec((TE, D), lambda i: (i, 0)),
            pl.BlockSpec((TE, D), lambda i: (i, 0)),
            pl.BlockSpec((TE, D), lambda i: (i, 0)),
            pl.BlockSpec((D, D), lambda i: (0, 0)),
            pl.BlockSpec((1, D), lambda i: (0, 0)),
            pl.BlockSpec((8, D), lambda i: (0, 0)),
            pl.BlockSpec((8, D), lambda i: (0, 0)),
            pl.BlockSpec((1, D), lambda i: (0, 0)),
            pl.BlockSpec((1, D), lambda i: (0, 0)),
        ],
        out_specs=(pl.BlockSpec((TE, D), lambda i: (i, 0)),
                   pl.BlockSpec((TE, D), lambda i: (i, 0))),
        out_shape=(out, out),
    )(edge_attr, gsum, gxv, WAT, bA, s1, s2, gamma_e, beta_e)


def _node_body(x_ref, xu_ref, agg_ref, gn_ref, bn_ref, xo_ref):
    node_in = xu_ref[...] + agg_ref[0] + agg_ref[1]
    mean = jnp.mean(node_in, axis=0, keepdims=True)
    var = jnp.mean(node_in * node_in, axis=0, keepdims=True) - mean * mean
    rstd = jax.lax.rsqrt(var + 1e-5)
    tmp = jnp.maximum(gn_ref[...] * (node_in - mean) * rstd + bn_ref[...], 0.0)
    xo_ref[...] = x_ref[...] + tmp


def _node_final(x, xu, agg, gamma_n, beta_n):
    return pl.pallas_call(
        _node_body,
        out_shape=jax.ShapeDtypeStruct((N, D), jnp.float32),
    )(x, xu, agg, gamma_n, beta_n)


def kernel(x, edge_index, edge_attr, Wu, bu, Wv, bv, WA, bA, WB, bB, WC, bC,
           gamma_node, beta_node, gamma_edge, beta_edge):
    row = edge_index[0]
    col = edge_index[1]
    bu2 = bu.reshape(1, D)
    bv2 = bv.reshape(1, D)
    bA2 = bA.reshape(1, D)
    bB2 = bB.reshape(1, D)
    bC2 = bC.reshape(1, D)
    ge2 = gamma_edge.reshape(1, D)
    be2 = beta_edge.reshape(1, D)
    gn2 = gamma_node.reshape(1, D)
    bn2 = beta_node.reshape(1, D)

    xu, xv, xB, xC = _tables(x, Wu.T, bu2, Wv.T, bv2, WB.T, bB2, WC.T, bC2)

    gsum, gxv = _sc_gather(xB, xC, xv, row, col)

    s1, s2 = _edge_stats(edge_attr, gsum, WA.T, bA2)
    eout, msg = _edge_apply(edge_attr, gsum, gxv, WA.T, bA2, s1, s2, ge2, be2)

    zeros_nd = jnp.zeros((N, D), jnp.float32)
    agg = _sc_scatter(msg, row, zeros_nd)

    x_out = _node_final(x, xu, agg, gn2, bn2)
    return (x_out, eout)
